# baseline (speedup 1.0000x reference)
import functools

import jax
import jax.numpy as jnp
from jax import lax
from jax.experimental import pallas as pl
from jax.experimental.pallas import tpu as pltpu
from jax.experimental.pallas import tpu_sc as plsc

_NC = 2
_NS = 16
_NW = _NC * _NS
_L = 16


def _depad_body(x_ref, o_ref):
    x = x_ref[...]
    r = x.reshape(x.shape[0] // 2, 2, x.shape[1])
    o_ref[...] = jnp.concatenate([r[:, 0, :], r[:, 1, :]], axis=1)


def _depad(x, blk):
    R, D = x.shape
    return pl.pallas_call(
        _depad_body,
        grid=(R // blk,),
        in_specs=[pl.BlockSpec((blk, D), lambda i: (i, 0))],
        out_specs=pl.BlockSpec((blk // 2, 2 * D), lambda i: (i, 0)),
        out_shape=jax.ShapeDtypeStruct((R // 2, 2 * D), x.dtype),
    )(x)


def _sc_scores(u_pos, v_pos, v_neg_flat, U, V, *, B, D, NNEG, CH):
    BPW = B // _NW
    NCH = BPW // CH
    ND = NNEG * CH
    NV = D // _L

    mesh = plsc.VectorSubcoreMesh(
        core_axis_name="c", subcore_axis_name="s",
        num_cores=_NC, num_subcores=_NS)

    def body(u_hbm, v_hbm, n_hbm, U_hbm, V_hbm, sp_hbm, sn_hbm,
             idx_u, idx_v, idx_n, pix_u, pix_v, pix_n,
             rows_u, rows_v, rows_n, sp_buf, sn_buf,
             tr_p, tr_q, sem):
        wid = lax.axis_index("s") * _NC + lax.axis_index("c")
        base = wid * BPW
        colbase = lax.iota(jnp.int32, 16) * 16

        def chunk(c, carry):
            off = base + c * CH
            pltpu.sync_copy(u_hbm.at[pl.ds(off, CH)], idx_u)
            pltpu.sync_copy(v_hbm.at[pl.ds(off, CH)], idx_v)
            pltpu.sync_copy(n_hbm.at[pl.ds(off * NNEG, ND)], idx_n)
            for t in range(CH // _L):
                s = pl.ds(t * _L, _L)
                pix_u[s] = lax.shift_right_logical(idx_u[s], 1)
                pix_v[s] = lax.shift_right_logical(idx_v[s], 1)
            for t in range(ND // _L):
                s = pl.ds(t * _L, _L)
                pix_n[s] = lax.shift_right_logical(idx_n[s], 1)
            cps = [pltpu.async_copy(U_hbm.at[pix_u], rows_u, sem),
                   pltpu.async_copy(V_hbm.at[pix_v], rows_v, sem)]
            for j in range(NNEG):
                cps.append(pltpu.async_copy(
                    V_hbm.at[pix_n.at[pl.ds(j * CH, CH)]],
                    rows_n.at[pl.ds(j * CH, CH)], sem))
            for cp in cps:
                cp.wait()

            def group(g, carry2):
                eb = g * _L
                pvu = (idx_u[pl.ds(eb, _L)] & 1) * D
                pvv = (idx_v[pl.ds(eb, _L)] & 1) * D
                pvn = [(idx_n[pl.ds(eb * NNEG + _L * t, _L)] & 1) * D
                       for t in range(NNEG)]
                for i in range(_L):
                    e = eb + i
                    ou = pvu[i]
                    ov = pvv[i]
                    u = [rows_u[e, pl.ds(ou + _L * j, _L)] for j in range(NV)]
                    v = [rows_v[e, pl.ds(ov + _L * j, _L)] for j in range(NV)]
                    p = u[0] * v[0]
                    for j in range(1, NV):
                        p = p + u[j] * v[j]
                    q = None
                    for k in range(NNEG):
                        fl = i * NNEG + k
                        on = pvn[fl // _L][fl % _L]
                        t = None
                        for j in range(NV):
                            r = rows_n[e * NNEG + k, pl.ds(on + _L * j, _L)]
                            t = u[j] * r if t is None else t + u[j] * r
                        q = t if q is None else q + t
                    tr_p[pl.ds(i * _L, _L)] = p
                    tr_q[pl.ds(i * _L, _L)] = q
                sp = None
                sn = None
                for j in range(_L):
                    tp = plsc.load_gather(tr_p, [colbase + j])
                    tq = plsc.load_gather(tr_q, [colbase + j])
                    sp = tp if sp is None else sp + tp
                    sn = tq if sn is None else sn + tq
                sp_buf[pl.ds(c * CH + eb, _L)] = sp * (1.0 / D)
                sn_buf[pl.ds(c * CH + eb, _L)] = sn * (1.0 / NNEG)
                return carry2

            lax.fori_loop(0, CH // _L, group, 0)
            return carry

        lax.fori_loop(0, NCH, chunk, 0)
        pltpu.sync_copy(sp_buf, sp_hbm.at[pl.ds(base, BPW)])
        pltpu.sync_copy(sn_buf, sn_hbm.at[pl.ds(base, BPW)])

    f = pl.kernel(
        body,
        out_type=[jax.ShapeDtypeStruct((B,), jnp.float32),
                  jax.ShapeDtypeStruct((B,), jnp.float32)],
        mesh=mesh,
        compiler_params=pltpu.CompilerParams(needs_layout_passes=False,
                                             use_tc_tiling_on_sc=False),
        scratch_types=[
            pltpu.VMEM((CH,), jnp.int32),
            pltpu.VMEM((CH,), jnp.int32),
            pltpu.VMEM((ND,), jnp.int32),
            pltpu.VMEM((CH,), jnp.int32),
            pltpu.VMEM((CH,), jnp.int32),
            pltpu.VMEM((ND,), jnp.int32),
            pltpu.VMEM((CH, 2 * D), jnp.float32),
            pltpu.VMEM((CH, 2 * D), jnp.float32),
            pltpu.VMEM((ND, 2 * D), jnp.float32),
            pltpu.VMEM((BPW,), jnp.float32),
            pltpu.VMEM((BPW,), jnp.float32),
            pltpu.VMEM((_L * _L,), jnp.float32),
            pltpu.VMEM((_L * _L,), jnp.float32),
            pltpu.SemaphoreType.DMA,
        ],
    )
    return f(u_pos, v_pos, v_neg_flat, U, V)


def _loss_body(sp_ref, sn_ref, o_ref, *, B):
    sp = sp_ref[...]
    sn = sn_ref[...]
    lt = jnp.minimum(sp, 0.0) - jnp.log1p(jnp.exp(-jnp.abs(sp)))
    ls = jnp.minimum(-sn, 0.0) - jnp.log1p(jnp.exp(-jnp.abs(sn)))
    o_ref[0, 0] = -(jnp.sum(lt) + jnp.sum(ls)) / B


def kernel(u_pos, v_pos, v_neg, batch_size, U, V):
    B = u_pos.shape[0]
    D = U.shape[1]
    NNEG = v_neg.shape[1]
    U1 = _depad(U, 8000)
    V1 = _depad(V, 8000)
    sp, sn = _sc_scores(u_pos, v_pos, v_neg.reshape(-1), U1, V1,
                        B=B, D=D, NNEG=NNEG, CH=128)
    loss = pl.pallas_call(
        functools.partial(_loss_body, B=B),
        out_shape=jax.ShapeDtypeStruct((1, 1), jnp.float32),
        in_specs=[pl.BlockSpec(memory_space=pltpu.VMEM),
                  pl.BlockSpec(memory_space=pltpu.VMEM)],
        out_specs=pl.BlockSpec(memory_space=pltpu.SMEM),
    )(sp.reshape(128, -1), sn.reshape(128, -1))
    return loss[0, 0]


# speedup vs baseline: 1.2998x; 1.2998x over previous
import functools

import jax
import jax.numpy as jnp
from jax import lax
from jax.experimental import pallas as pl
from jax.experimental.pallas import tpu as pltpu
from jax.experimental.pallas import tpu_sc as plsc

_NC = 2
_NS = 16
_NW = _NC * _NS
_L = 16


def _depad_body(x_ref, o_ref):
    x = x_ref[...]
    r = x.reshape(x.shape[0] // 2, 2, x.shape[1])
    o_ref[...] = jnp.concatenate([r[:, 0, :], r[:, 1, :]], axis=1)


def _depad(x, blk):
    R, D = x.shape
    return pl.pallas_call(
        _depad_body,
        grid=(R // blk,),
        in_specs=[pl.BlockSpec((blk, D), lambda i: (i, 0))],
        out_specs=pl.BlockSpec((blk // 2, 2 * D), lambda i: (i, 0)),
        out_shape=jax.ShapeDtypeStruct((R // 2, 2 * D), x.dtype),
    )(x)


def _sc_scores(u_pos, v_pos, v_neg_flat, U, V, *, B, D, NNEG, CH):
    BPW = B // _NW
    NCH = BPW // CH
    ND = NNEG * CH
    NV = D // _L

    mesh = plsc.VectorSubcoreMesh(
        core_axis_name="c", subcore_axis_name="s",
        num_cores=_NC, num_subcores=_NS)

    def body(u_hbm, v_hbm, n_hbm, U_hbm, V_hbm, sp_hbm, sn_hbm,
             idx_u, idx_v, idx_n, pix_u, pix_v, pix_n,
             rows_u, rows_v, rows_n, sp_buf, sn_buf,
             tr_p, tr_q, sem):
        wid = lax.axis_index("s") * _NC + lax.axis_index("c")
        base = wid * BPW
        colbase = lax.iota(jnp.int32, 16) * 16

        def chunk(c, carry):
            off = base + c * CH
            pltpu.sync_copy(u_hbm.at[pl.ds(off, CH)], idx_u)
            pltpu.sync_copy(v_hbm.at[pl.ds(off, CH)], idx_v)
            pltpu.sync_copy(n_hbm.at[pl.ds(off * NNEG, ND)], idx_n)
            for t in range(CH // _L):
                s = pl.ds(t * _L, _L)
                pix_u[s] = lax.shift_right_logical(idx_u[s], 1)
                pix_v[s] = lax.shift_right_logical(idx_v[s], 1)
            for t in range(ND // _L):
                s = pl.ds(t * _L, _L)
                pix_n[s] = lax.shift_right_logical(idx_n[s], 1)
            cps = [pltpu.async_copy(U_hbm.at[pix_u], rows_u, sem),
                   pltpu.async_copy(V_hbm.at[pix_v], rows_v, sem)]
            for j in range(NNEG):
                cps.append(pltpu.async_copy(
                    V_hbm.at[pix_n.at[pl.ds(j * CH, CH)]],
                    rows_n.at[pl.ds(j * CH, CH)], sem))
            for cp in cps:
                cp.wait()

            def group(g, carry2):
                eb = g * _L
                pvu = (idx_u[pl.ds(eb, _L)] & 1) * D
                pvv = (idx_v[pl.ds(eb, _L)] & 1) * D
                pvn = [(idx_n[pl.ds(eb * NNEG + _L * t, _L)] & 1) * D
                       for t in range(NNEG)]
                for i in range(_L):
                    e = eb + i
                    ou = pvu[i]
                    ov = pvv[i]
                    u = [rows_u[e, pl.ds(ou + _L * j, _L)] for j in range(NV)]
                    v = [rows_v[e, pl.ds(ov + _L * j, _L)] for j in range(NV)]
                    p = u[0] * v[0]
                    for j in range(1, NV):
                        p = p + u[j] * v[j]
                    q = None
                    for k in range(NNEG):
                        fl = i * NNEG + k
                        on = pvn[fl // _L][fl % _L]
                        t = None
                        for j in range(NV):
                            r = rows_n[e * NNEG + k, pl.ds(on + _L * j, _L)]
                            t = u[j] * r if t is None else t + u[j] * r
                        q = t if q is None else q + t
                    tr_p[pl.ds(i * _L, _L)] = p
                    tr_q[pl.ds(i * _L, _L)] = q
                sp = None
                sn = None
                for j in range(_L):
                    tp = plsc.load_gather(tr_p, [colbase + j])
                    tq = plsc.load_gather(tr_q, [colbase + j])
                    sp = tp if sp is None else sp + tp
                    sn = tq if sn is None else sn + tq
                sp_buf[pl.ds(c * CH + eb, _L)] = sp * (1.0 / D)
                sn_buf[pl.ds(c * CH + eb, _L)] = sn * (1.0 / NNEG)
                return carry2

            lax.fori_loop(0, CH // _L, group, 0)
            return carry

        lax.fori_loop(0, NCH, chunk, 0)
        pltpu.sync_copy(sp_buf, sp_hbm.at[pl.ds(base, BPW)])
        pltpu.sync_copy(sn_buf, sn_hbm.at[pl.ds(base, BPW)])

    f = pl.kernel(
        body,
        out_type=[jax.ShapeDtypeStruct((B,), jnp.float32),
                  jax.ShapeDtypeStruct((B,), jnp.float32)],
        mesh=mesh,
        compiler_params=pltpu.CompilerParams(needs_layout_passes=False,
                                             use_tc_tiling_on_sc=False),
        scratch_types=[
            pltpu.VMEM((CH,), jnp.int32),
            pltpu.VMEM((CH,), jnp.int32),
            pltpu.VMEM((ND,), jnp.int32),
            pltpu.VMEM((CH,), jnp.int32),
            pltpu.VMEM((CH,), jnp.int32),
            pltpu.VMEM((ND,), jnp.int32),
            pltpu.VMEM((CH, 2 * D), jnp.float32),
            pltpu.VMEM((CH, 2 * D), jnp.float32),
            pltpu.VMEM((ND, 2 * D), jnp.float32),
            pltpu.VMEM((BPW,), jnp.float32),
            pltpu.VMEM((BPW,), jnp.float32),
            pltpu.VMEM((_L * _L,), jnp.float32),
            pltpu.VMEM((_L * _L,), jnp.float32),
            pltpu.SemaphoreType.DMA,
        ],
    )
    return f(u_pos, v_pos, v_neg_flat, U, V)


def _loss_body(sp_ref, sn_ref, o_ref, *, B):
    sp = sp_ref[...]
    sn = sn_ref[...]
    lt = jnp.minimum(sp, 0.0) - jnp.log1p(jnp.exp(-jnp.abs(sp)))
    ls = jnp.minimum(-sn, 0.0) - jnp.log1p(jnp.exp(-jnp.abs(sn)))
    o_ref[0, 0] = -(jnp.sum(lt) + jnp.sum(ls)) / B


def kernel(u_pos, v_pos, v_neg, batch_size, U, V):
    B = u_pos.shape[0]
    D = U.shape[1]
    NNEG = v_neg.shape[1]
    U1 = U.reshape(U.shape[0] // 2, 2 * D)
    V1 = V.reshape(V.shape[0] // 2, 2 * D)
    sp, sn = _sc_scores(u_pos, v_pos, v_neg.reshape(-1), U1, V1,
                        B=B, D=D, NNEG=NNEG, CH=128)
    loss = pl.pallas_call(
        functools.partial(_loss_body, B=B),
        out_shape=jax.ShapeDtypeStruct((1, 1), jnp.float32),
        in_specs=[pl.BlockSpec(memory_space=pltpu.VMEM),
                  pl.BlockSpec(memory_space=pltpu.VMEM)],
        out_specs=pl.BlockSpec(memory_space=pltpu.SMEM),
    )(sp.reshape(128, -1), sn.reshape(128, -1))
    return loss[0, 0]


# speedup vs baseline: 2.2164x; 1.7052x over previous
import functools

import jax
import jax.numpy as jnp
from jax import lax
from jax.experimental import pallas as pl
from jax.experimental.pallas import tpu as pltpu
from jax.experimental.pallas import tpu_sc as plsc

_NC = 2
_NS = 16
_NW = _NC * _NS
_L = 16
_SL = 8


def _sc_scores(u_pos, v_pos, v_neg_flat, U3, V3, *, B, D, NNEG, CH):
    BPW = B // _NW
    NCH = BPW // CH
    ND = NNEG * CH
    NV = D // _L

    mesh = plsc.VectorSubcoreMesh(
        core_axis_name="c", subcore_axis_name="s",
        num_cores=_NC, num_subcores=_NS)

    def body(u_hbm, v_hbm, n_hbm, U_hbm, V_hbm, sp_hbm, sn_hbm,
             idx_u, idx_v, idx_n,
             rows_u, rows_v, rows_n, sp_buf, sn_buf, tr_p, tr_q, sem):
        wid = lax.axis_index("s") * _NC + lax.axis_index("c")
        base = wid * BPW
        colbase = lax.iota(jnp.int32, 16) * 16

        def chunk(c, carry):
            off = base + c * CH
            pltpu.sync_copy(u_hbm.at[pl.ds(off, CH)], idx_u)
            pltpu.sync_copy(v_hbm.at[pl.ds(off, CH)], idx_v)
            pltpu.sync_copy(n_hbm.at[pl.ds(off * NNEG, ND)], idx_n)

            def fire_pos(t, cr):
                pu = lax.shift_right_logical(idx_u[pl.ds(t * _L, _L)], 3)
                pv = lax.shift_right_logical(idx_v[pl.ds(t * _L, _L)], 3)
                for i in range(_L):
                    e = t * _L + i
                    pltpu.async_copy(U_hbm.at[pu[i]], rows_u.at[e], sem)
                    pltpu.async_copy(V_hbm.at[pv[i]], rows_v.at[e], sem)
                return cr

            lax.fori_loop(0, CH // _L, fire_pos, 0)

            def fire_neg(t, cr):
                pn = lax.shift_right_logical(idx_n[pl.ds(t * _L, _L)], 3)
                for i in range(_L):
                    e = t * _L + i
                    pltpu.async_copy(V_hbm.at[pn[i]], rows_n.at[e], sem)
                return cr

            lax.fori_loop(0, ND // _L, fire_neg, 0)

            pltpu.make_async_copy(U_hbm.at[pl.ds(0, CH)], rows_u, sem).wait()
            pltpu.make_async_copy(U_hbm.at[pl.ds(0, CH)], rows_v, sem).wait()
            pltpu.make_async_copy(U_hbm.at[pl.ds(0, ND)], rows_n, sem).wait()

            def group(g, carry2):
                eb = g * _L
                svu = idx_u[pl.ds(eb, _L)] & 7
                svv = idx_v[pl.ds(eb, _L)] & 7
                svn = [idx_n[pl.ds(eb * NNEG + _L * t, _L)] & 7
                       for t in range(NNEG)]
                for i in range(_L):
                    e = eb + i
                    su = svu[i]
                    sv = svv[i]
                    u = [rows_u[e, su, pl.ds(_L * j, _L)] for j in range(NV)]
                    v = [rows_v[e, sv, pl.ds(_L * j, _L)] for j in range(NV)]
                    p = u[0] * v[0]
                    for j in range(1, NV):
                        p = p + u[j] * v[j]
                    q = None
                    for k in range(NNEG):
                        fl = i * NNEG + k
                        sn_ = svn[fl // _L][fl % _L]
                        t = None
                        for j in range(NV):
                            r = rows_n[e * NNEG + k, sn_, pl.ds(_L * j, _L)]
                            t = u[j] * r if t is None else t + u[j] * r
                        q = t if q is None else q + t
                    tr_p[pl.ds(i * _L, _L)] = p
                    tr_q[pl.ds(i * _L, _L)] = q
                sp = None
                sn = None
                for j in range(_L):
                    tp = plsc.load_gather(tr_p, [colbase + j])
                    tq = plsc.load_gather(tr_q, [colbase + j])
                    sp = tp if sp is None else sp + tp
                    sn = tq if sn is None else sn + tq
                sp_buf[pl.ds(c * CH + eb, _L)] = sp * (1.0 / D)
                sn_buf[pl.ds(c * CH + eb, _L)] = sn * (1.0 / NNEG)
                return carry2

            lax.fori_loop(0, CH // _L, group, 0)
            return carry

        lax.fori_loop(0, NCH, chunk, 0)
        pltpu.sync_copy(sp_buf, sp_hbm.at[pl.ds(base, BPW)])
        pltpu.sync_copy(sn_buf, sn_hbm.at[pl.ds(base, BPW)])

    f = pl.kernel(
        body,
        out_type=[jax.ShapeDtypeStruct((B,), jnp.float32),
                  jax.ShapeDtypeStruct((B,), jnp.float32)],
        mesh=mesh,
        compiler_params=pltpu.CompilerParams(needs_layout_passes=False,
                                             use_tc_tiling_on_sc=True),
        scratch_types=[
            pltpu.VMEM((CH,), jnp.int32),
            pltpu.VMEM((CH,), jnp.int32),
            pltpu.VMEM((ND,), jnp.int32),
            pltpu.VMEM((CH, _SL, D), jnp.float32),
            pltpu.VMEM((CH, _SL, D), jnp.float32),
            pltpu.VMEM((ND, _SL, D), jnp.float32),
            pltpu.VMEM((BPW,), jnp.float32),
            pltpu.VMEM((BPW,), jnp.float32),
            pltpu.VMEM((_L * _L,), jnp.float32),
            pltpu.VMEM((_L * _L,), jnp.float32),
            pltpu.SemaphoreType.DMA,
        ],
    )
    return f(u_pos, v_pos, v_neg_flat, U3, V3)


def _loss_body(sp_ref, sn_ref, o_ref, *, B):
    sp = sp_ref[...]
    sn = sn_ref[...]
    lt = jnp.minimum(sp, 0.0) - jnp.log1p(jnp.exp(-jnp.abs(sp)))
    ls = jnp.minimum(-sn, 0.0) - jnp.log1p(jnp.exp(-jnp.abs(sn)))
    o_ref[0, 0] = -(jnp.sum(lt) + jnp.sum(ls)) / B


def kernel(u_pos, v_pos, v_neg, batch_size, U, V):
    B = u_pos.shape[0]
    D = U.shape[1]
    NNEG = v_neg.shape[1]
    U3 = U.reshape(U.shape[0] // _SL, _SL, D)
    V3 = V.reshape(V.shape[0] // _SL, _SL, D)
    sp, sn = _sc_scores(u_pos, v_pos, v_neg.reshape(-1), U3, V3,
                        B=B, D=D, NNEG=NNEG, CH=16)
    loss = pl.pallas_call(
        functools.partial(_loss_body, B=B),
        out_shape=jax.ShapeDtypeStruct((1, 1), jnp.float32),
        in_specs=[pl.BlockSpec(memory_space=pltpu.VMEM),
                  pl.BlockSpec(memory_space=pltpu.VMEM)],
        out_specs=pl.BlockSpec(memory_space=pltpu.SMEM),
    )(sp.reshape(128, -1), sn.reshape(128, -1))
    return loss[0, 0]
